# baseline (speedup 1.0000x reference)
import functools

import jax
import jax.numpy as jnp
from jax import lax
from jax.experimental import pallas as pl
from jax.experimental.pallas import tpu as pltpu
from jax.experimental.pallas import tpu_sc as plsc

N_NODES = 10000
N_EDGES = 320000
D_IN = 128
D_OUT = 128

NC = 2
NS = 16

N_PAD = 10240
ROWS_PER_TILE = N_PAD // NS
DST_SENTINEL = 10200

EPT = 10000
CHUNK = 64
CPT = EPT // CHUNK
TAIL = EPT - CPT * CHUNK
NBUF = 3

MM_BLOCK = 1000
CB_BLOCK = 1000


def _final_body(p_ref, w_ref, b_ref, out_ref):
    out_ref[...] = jnp.dot(p_ref[0] + p_ref[1], w_ref[...],
                           preferred_element_type=jnp.float32,
                           precision=lax.Precision.HIGHEST) + b_ref[...]


def _final(partials, W, b2d):
    return pl.pallas_call(
        _final_body,
        grid=(N_NODES // CB_BLOCK,),
        in_specs=[
            pl.BlockSpec((NC, CB_BLOCK, D_IN), lambda i: (0, i, 0)),
            pl.BlockSpec((D_IN, D_OUT), lambda i: (0, 0)),
            pl.BlockSpec((1, D_OUT), lambda i: (0, 0)),
        ],
        out_specs=pl.BlockSpec((CB_BLOCK, D_OUT), lambda i: (i, 0)),
        out_shape=jax.ShapeDtypeStruct((N_NODES, D_OUT), jnp.float32),
    )(partials, W, b2d)


def _sc_body(h_hbm, src_hbm, dst_hbm, part_hbm,
             sall, dall, r0, r1, r2, acc_sh, g0, g1, g2, s0, s1, s2, isem):
    rows = (r0, r1, r2)[:NBUF]
    gsem = (g0, g1, g2)[:NBUF]
    ssem = (s0, s1, s2)[:NBUF]
    cid = lax.axis_index("c")
    sid = lax.axis_index("s")
    wid = cid * NS + sid
    row0 = sid * ROWS_PER_TILE
    ebase = wid * EPT

    pltpu.async_copy(src_hbm.at[pl.ds(ebase, EPT)], sall, isem)
    pltpu.async_copy(dst_hbm.at[pl.ds(ebase, EPT)], dall, isem)

    zbuf = rows[0]
    zvec = jnp.zeros((16,), jnp.float32)

    def _zfill(t, _):
        zbuf[t // 8, pl.ds((t % 8) * 16, 16)] = zvec
        return 0

    lax.fori_loop(0, CHUNK * (D_OUT // 16), _zfill, 0)
    for k in range(ROWS_PER_TILE // CHUNK):
        pltpu.sync_copy(zbuf, acc_sh.at[pl.ds(row0 + k * CHUNK, CHUNK)])

    pltpu.make_async_copy(src_hbm.at[pl.ds(ebase, EPT)], sall, isem).wait()
    pltpu.make_async_copy(dst_hbm.at[pl.ds(ebase, EPT)], dall, isem).wait()
    plsc.subcore_barrier()

    def _sl(g):
        return pl.ds(lax.mul(g, CHUNK), CHUNK)

    def _gather(g, b):
        pltpu.async_copy(h_hbm.at[sall.at[_sl(g)]], rows[b], gsem[b])

    for b in range(NBUF):
        _gather(b, b)

    def _proc(g, b, refill):
        pltpu.make_async_copy(
            h_hbm.at[sall.at[_sl(g)]], rows[b], gsem[b]).wait()
        pltpu.async_copy(rows[b], acc_sh.at[dall.at[_sl(g)]], ssem[b],
                         add=True)
        if refill:
            @pl.when(g + NBUF < CPT)
            def _():
                pltpu.make_async_copy(
                    rows[b], acc_sh.at[dall.at[_sl(g)]], ssem[b]).wait()
                _gather(g + NBUF, b)
        else:
            pltpu.make_async_copy(
                rows[b], acc_sh.at[dall.at[_sl(g)]], ssem[b]).wait()

    def _step(t, _):
        for b in range(NBUF):
            _proc(t * NBUF + b, b, True)
        return 0

    lax.fori_loop(0, CPT // NBUF, _step, 0)
    for g in range((CPT // NBUF) * NBUF, CPT):
        _proc(g, g % NBUF, False)
    for g in range(max(0, CPT - NBUF), (CPT // NBUF) * NBUF):
        b = g % NBUF
        pltpu.make_async_copy(
            rows[b], acc_sh.at[dall.at[_sl(g)]], ssem[b]).wait()
    if TAIL:
        toff = pl.ds(CPT * CHUNK, TAIL)
        tbuf = rows[0].at[pl.ds(0, TAIL)]
        pltpu.async_copy(h_hbm.at[sall.at[toff]], tbuf, gsem[0])
        pltpu.make_async_copy(h_hbm.at[sall.at[toff]], tbuf, gsem[0]).wait()
        pltpu.sync_copy(tbuf, acc_sh.at[dall.at[toff]], add=True)
    plsc.subcore_barrier()

    pltpu.sync_copy(
        acc_sh.at[pl.ds(row0, ROWS_PER_TILE)],
        part_hbm.at[cid, pl.ds(row0, ROWS_PER_TILE)],
    )


_sc_aggregate = functools.partial(
    pl.kernel,
    out_type=jax.ShapeDtypeStruct((NC, N_PAD, D_OUT), jnp.float32),
    mesh=plsc.VectorSubcoreMesh(core_axis_name="c", subcore_axis_name="s"),
    scratch_types=[
        pltpu.VMEM((EPT,), jnp.int32),
        pltpu.VMEM((EPT,), jnp.int32),
        pltpu.VMEM((CHUNK, D_OUT), jnp.float32),
        pltpu.VMEM((CHUNK, D_OUT), jnp.float32),
        pltpu.VMEM((CHUNK, D_OUT), jnp.float32),
        pltpu.VMEM_SHARED((N_PAD, D_OUT), jnp.float32),
        pltpu.SemaphoreType.DMA,
        pltpu.SemaphoreType.DMA,
        pltpu.SemaphoreType.DMA,
        pltpu.SemaphoreType.DMA,
        pltpu.SemaphoreType.DMA,
        pltpu.SemaphoreType.DMA,
        pltpu.SemaphoreType.DMA,
    ],
)(_sc_body)


def kernel(x, adj, W, b):
    partials = _sc_aggregate(x, adj[0], adj[1])
    return _final(partials, W, b.reshape(1, D_OUT))


# speedup vs baseline: 1.0668x; 1.0668x over previous
import functools

import jax
import jax.numpy as jnp
from jax import lax
from jax.experimental import pallas as pl
from jax.experimental.pallas import tpu as pltpu
from jax.experimental.pallas import tpu_sc as plsc

N_NODES = 10000
N_EDGES = 320000
D_IN = 128
D_OUT = 128

NC = 2
NS = 16

N_PAD = 10240
ROWS_PER_TILE = N_PAD // NS
DST_SENTINEL = 10200

EPT = 10000
CHUNK = 48
CPT = EPT // CHUNK
TAIL = EPT - CPT * CHUNK
NBUF = 4

MM_BLOCK = 1000
CB_BLOCK = 1000


def _final_body(p_ref, w_ref, b_ref, out_ref):
    out_ref[...] = jnp.dot(p_ref[0] + p_ref[1], w_ref[...],
                           preferred_element_type=jnp.float32) + b_ref[...]


def _final(partials, W, b2d):
    return pl.pallas_call(
        _final_body,
        grid=(N_NODES // CB_BLOCK,),
        in_specs=[
            pl.BlockSpec((NC, CB_BLOCK, D_IN), lambda i: (0, i, 0)),
            pl.BlockSpec((D_IN, D_OUT), lambda i: (0, 0)),
            pl.BlockSpec((1, D_OUT), lambda i: (0, 0)),
        ],
        out_specs=pl.BlockSpec((CB_BLOCK, D_OUT), lambda i: (i, 0)),
        out_shape=jax.ShapeDtypeStruct((N_NODES, D_OUT), jnp.float32),
    )(partials, W, b2d)


def _sc_body(h_hbm, src_hbm, dst_hbm, part_hbm,
             sall, dall, r0, r1, r2, r3, acc_sh,
             g0, g1, g2, g3, s0, s1, s2, s3, isem):
    rows = (r0, r1, r2, r3)[:NBUF]
    gsem = (g0, g1, g2, g3)[:NBUF]
    ssem = (s0, s1, s2, s3)[:NBUF]
    cid = lax.axis_index("c")
    sid = lax.axis_index("s")
    wid = cid * NS + sid
    row0 = sid * ROWS_PER_TILE
    ebase = wid * EPT

    pltpu.async_copy(src_hbm.at[pl.ds(ebase, EPT)], sall, isem)
    pltpu.async_copy(dst_hbm.at[pl.ds(ebase, EPT)], dall, isem)

    zbuf = rows[0]
    zvec = jnp.zeros((16,), jnp.float32)

    def _zfill(t, _):
        zbuf[t // 8, pl.ds((t % 8) * 16, 16)] = zvec
        return 0

    lax.fori_loop(0, CHUNK * (D_OUT // 16), _zfill, 0)
    for k in range(ROWS_PER_TILE // CHUNK):
        pltpu.sync_copy(zbuf, acc_sh.at[pl.ds(row0 + k * CHUNK, CHUNK)])
    zrem = ROWS_PER_TILE % CHUNK
    if zrem:
        pltpu.sync_copy(
            zbuf.at[pl.ds(0, zrem)],
            acc_sh.at[pl.ds(row0 + (ROWS_PER_TILE // CHUNK) * CHUNK, zrem)])

    pltpu.make_async_copy(src_hbm.at[pl.ds(ebase, EPT)], sall, isem).wait()
    pltpu.make_async_copy(dst_hbm.at[pl.ds(ebase, EPT)], dall, isem).wait()
    plsc.subcore_barrier()

    def _sl(g):
        return pl.ds(lax.mul(g, CHUNK), CHUNK)

    def _gather(g, b):
        pltpu.async_copy(h_hbm.at[sall.at[_sl(g)]], rows[b], gsem[b])

    for b in range(NBUF):
        _gather(b, b)

    def _proc(g, b, refill):
        pltpu.make_async_copy(
            h_hbm.at[sall.at[_sl(g)]], rows[b], gsem[b]).wait()
        pltpu.async_copy(rows[b], acc_sh.at[dall.at[_sl(g)]], ssem[b],
                         add=True)
        if refill:
            @pl.when(g + NBUF < CPT)
            def _():
                pltpu.make_async_copy(
                    rows[b], acc_sh.at[dall.at[_sl(g)]], ssem[b]).wait()
                _gather(g + NBUF, b)
        else:
            pltpu.make_async_copy(
                rows[b], acc_sh.at[dall.at[_sl(g)]], ssem[b]).wait()

    def _step(t, _):
        for b in range(NBUF):
            _proc(t * NBUF + b, b, True)
        return 0

    lax.fori_loop(0, CPT // NBUF, _step, 0)
    for g in range((CPT // NBUF) * NBUF, CPT):
        _proc(g, g % NBUF, False)
    for g in range(max(0, CPT - NBUF), (CPT // NBUF) * NBUF):
        b = g % NBUF
        pltpu.make_async_copy(
            rows[b], acc_sh.at[dall.at[_sl(g)]], ssem[b]).wait()
    if TAIL:
        toff = pl.ds(CPT * CHUNK, TAIL)
        tbuf = rows[0].at[pl.ds(0, TAIL)]
        pltpu.async_copy(h_hbm.at[sall.at[toff]], tbuf, gsem[0])
        pltpu.make_async_copy(h_hbm.at[sall.at[toff]], tbuf, gsem[0]).wait()
        pltpu.sync_copy(tbuf, acc_sh.at[dall.at[toff]], add=True)
    plsc.subcore_barrier()

    pltpu.sync_copy(
        acc_sh.at[pl.ds(row0, ROWS_PER_TILE)],
        part_hbm.at[cid, pl.ds(row0, ROWS_PER_TILE)],
    )


_sc_aggregate = functools.partial(
    pl.kernel,
    out_type=jax.ShapeDtypeStruct((NC, N_PAD, D_OUT), jnp.float32),
    mesh=plsc.VectorSubcoreMesh(core_axis_name="c", subcore_axis_name="s"),
    scratch_types=[
        pltpu.VMEM((EPT,), jnp.int32),
        pltpu.VMEM((EPT,), jnp.int32),
        pltpu.VMEM((CHUNK, D_OUT), jnp.float32),
        pltpu.VMEM((CHUNK, D_OUT), jnp.float32),
        pltpu.VMEM((CHUNK, D_OUT), jnp.float32),
        pltpu.VMEM((CHUNK, D_OUT), jnp.float32),
        pltpu.VMEM_SHARED((N_PAD, D_OUT), jnp.float32),
        pltpu.SemaphoreType.DMA,
        pltpu.SemaphoreType.DMA,
        pltpu.SemaphoreType.DMA,
        pltpu.SemaphoreType.DMA,
        pltpu.SemaphoreType.DMA,
        pltpu.SemaphoreType.DMA,
        pltpu.SemaphoreType.DMA,
        pltpu.SemaphoreType.DMA,
        pltpu.SemaphoreType.DMA,
    ],
)(_sc_body)


def kernel(x, adj, W, b):
    partials = _sc_aggregate(x, adj[0], adj[1])
    return _final(partials, W, b.reshape(1, D_OUT))


# speedup vs baseline: 1.0784x; 1.0108x over previous
import functools

import jax
import jax.numpy as jnp
from jax import lax
from jax.experimental import pallas as pl
from jax.experimental.pallas import tpu as pltpu
from jax.experimental.pallas import tpu_sc as plsc

N_NODES = 10000
N_EDGES = 320000
D_IN = 128
D_OUT = 128

NC = 2
NS = 16

N_PAD = 10240
ROWS_PER_TILE = N_PAD // NS
DST_SENTINEL = 10200

EPT = 10000
CHUNK = 56
CPT = EPT // CHUNK
TAIL = EPT - CPT * CHUNK
NBUF = 4

MM_BLOCK = 1000
CB_BLOCK = 1000


def _final_body(p_ref, w_ref, b_ref, out_ref):
    out_ref[...] = jnp.dot(p_ref[0] + p_ref[1], w_ref[...],
                           preferred_element_type=jnp.float32) + b_ref[...]


def _final(partials, W, b2d):
    return pl.pallas_call(
        _final_body,
        grid=(N_NODES // CB_BLOCK,),
        in_specs=[
            pl.BlockSpec((NC, CB_BLOCK, D_IN), lambda i: (0, i, 0)),
            pl.BlockSpec((D_IN, D_OUT), lambda i: (0, 0)),
            pl.BlockSpec((1, D_OUT), lambda i: (0, 0)),
        ],
        out_specs=pl.BlockSpec((CB_BLOCK, D_OUT), lambda i: (i, 0)),
        out_shape=jax.ShapeDtypeStruct((N_NODES, D_OUT), jnp.float32),
    )(partials, W, b2d)


def _sc_body(h_hbm, src_hbm, dst_hbm, part_hbm,
             sall, dall, r0, r1, r2, r3, acc_sh,
             g0, g1, g2, g3, s0, s1, s2, s3, isem):
    rows = (r0, r1, r2, r3)[:NBUF]
    gsem = (g0, g1, g2, g3)[:NBUF]
    ssem = (s0, s1, s2, s3)[:NBUF]
    cid = lax.axis_index("c")
    sid = lax.axis_index("s")
    wid = cid * NS + sid
    row0 = sid * ROWS_PER_TILE
    ebase = wid * EPT

    pltpu.async_copy(src_hbm.at[pl.ds(ebase, EPT)], sall, isem)
    pltpu.async_copy(dst_hbm.at[pl.ds(ebase, EPT)], dall, isem)

    zbuf = rows[0]
    zvec = jnp.zeros((16,), jnp.float32)

    def _zfill(t, _):
        zbuf[t // 8, pl.ds((t % 8) * 16, 16)] = zvec
        return 0

    lax.fori_loop(0, CHUNK * (D_OUT // 16), _zfill, 0)
    for k in range(ROWS_PER_TILE // CHUNK):
        pltpu.sync_copy(zbuf, acc_sh.at[pl.ds(row0 + k * CHUNK, CHUNK)])
    zrem = ROWS_PER_TILE % CHUNK
    if zrem:
        pltpu.sync_copy(
            zbuf.at[pl.ds(0, zrem)],
            acc_sh.at[pl.ds(row0 + (ROWS_PER_TILE // CHUNK) * CHUNK, zrem)])

    pltpu.make_async_copy(src_hbm.at[pl.ds(ebase, EPT)], sall, isem).wait()
    pltpu.make_async_copy(dst_hbm.at[pl.ds(ebase, EPT)], dall, isem).wait()
    plsc.subcore_barrier()

    def _sl(g):
        return pl.ds(lax.mul(g, CHUNK), CHUNK)

    def _gather(g, b):
        pltpu.async_copy(h_hbm.at[sall.at[_sl(g)]], rows[b], gsem[b])

    for b in range(NBUF):
        _gather(b, b)

    def _proc(g, b, refill):
        pltpu.make_async_copy(
            h_hbm.at[sall.at[_sl(g)]], rows[b], gsem[b]).wait()
        pltpu.async_copy(rows[b], acc_sh.at[dall.at[_sl(g)]], ssem[b],
                         add=True)
        if refill:
            @pl.when(g + NBUF < CPT)
            def _():
                pltpu.make_async_copy(
                    rows[b], acc_sh.at[dall.at[_sl(g)]], ssem[b]).wait()
                _gather(g + NBUF, b)
        else:
            pltpu.make_async_copy(
                rows[b], acc_sh.at[dall.at[_sl(g)]], ssem[b]).wait()

    def _step(t, _):
        for b in range(NBUF):
            _proc(t * NBUF + b, b, True)
        return 0

    lax.fori_loop(0, CPT // NBUF, _step, 0)
    for g in range((CPT // NBUF) * NBUF, CPT):
        _proc(g, g % NBUF, False)
    for g in range(max(0, CPT - NBUF), (CPT // NBUF) * NBUF):
        b = g % NBUF
        pltpu.make_async_copy(
            rows[b], acc_sh.at[dall.at[_sl(g)]], ssem[b]).wait()
    if TAIL:
        toff = pl.ds(CPT * CHUNK, TAIL)
        tbuf = rows[0].at[pl.ds(0, TAIL)]
        pltpu.async_copy(h_hbm.at[sall.at[toff]], tbuf, gsem[0])
        pltpu.make_async_copy(h_hbm.at[sall.at[toff]], tbuf, gsem[0]).wait()
        pltpu.sync_copy(tbuf, acc_sh.at[dall.at[toff]], add=True)
    plsc.subcore_barrier()

    pltpu.sync_copy(
        acc_sh.at[pl.ds(row0, ROWS_PER_TILE)],
        part_hbm.at[cid, pl.ds(row0, ROWS_PER_TILE)],
    )


_sc_aggregate = functools.partial(
    pl.kernel,
    out_type=jax.ShapeDtypeStruct((NC, N_PAD, D_OUT), jnp.float32),
    mesh=plsc.VectorSubcoreMesh(core_axis_name="c", subcore_axis_name="s"),
    scratch_types=[
        pltpu.VMEM((EPT,), jnp.int32),
        pltpu.VMEM((EPT,), jnp.int32),
        pltpu.VMEM((CHUNK, D_OUT), jnp.float32),
        pltpu.VMEM((CHUNK, D_OUT), jnp.float32),
        pltpu.VMEM((CHUNK, D_OUT), jnp.float32),
        pltpu.VMEM((CHUNK, D_OUT), jnp.float32),
        pltpu.VMEM_SHARED((N_PAD, D_OUT), jnp.float32),
        pltpu.SemaphoreType.DMA,
        pltpu.SemaphoreType.DMA,
        pltpu.SemaphoreType.DMA,
        pltpu.SemaphoreType.DMA,
        pltpu.SemaphoreType.DMA,
        pltpu.SemaphoreType.DMA,
        pltpu.SemaphoreType.DMA,
        pltpu.SemaphoreType.DMA,
        pltpu.SemaphoreType.DMA,
    ],
)(_sc_body)


def kernel(x, adj, W, b):
    partials = _sc_aggregate(x, adj[0], adj[1])
    return _final(partials, W, b.reshape(1, D_OUT))


# speedup vs baseline: 1.0891x; 1.0100x over previous
import functools

import jax
import jax.numpy as jnp
from jax import lax
from jax.experimental import pallas as pl
from jax.experimental.pallas import tpu as pltpu
from jax.experimental.pallas import tpu_sc as plsc

N_NODES = 10000
N_EDGES = 320000
D_IN = 128
D_OUT = 128

NC = 2
NS = 16

N_PAD = 10240
ROWS_PER_TILE = N_PAD // NS
DST_SENTINEL = 10200

EPT = 10000
CHUNK = 40
CPT = EPT // CHUNK
TAIL = EPT - CPT * CHUNK
NBUF = 5

MM_BLOCK = 1000
CB_BLOCK = 1000


def _final_body(p_ref, w_ref, b_ref, out_ref):
    out_ref[...] = jnp.dot(p_ref[0] + p_ref[1], w_ref[...],
                           preferred_element_type=jnp.float32) + b_ref[...]


def _final(partials, W, b2d):
    return pl.pallas_call(
        _final_body,
        grid=(N_NODES // CB_BLOCK,),
        in_specs=[
            pl.BlockSpec((NC, CB_BLOCK, D_IN), lambda i: (0, i, 0)),
            pl.BlockSpec((D_IN, D_OUT), lambda i: (0, 0)),
            pl.BlockSpec((1, D_OUT), lambda i: (0, 0)),
        ],
        out_specs=pl.BlockSpec((CB_BLOCK, D_OUT), lambda i: (i, 0)),
        out_shape=jax.ShapeDtypeStruct((N_NODES, D_OUT), jnp.float32),
    )(partials, W, b2d)


def _sc_body(h_hbm, src_hbm, dst_hbm, part_hbm,
             sall, dall, r0, r1, r2, r3, r4, acc_sh,
             g0, g1, g2, g3, g4, s0, s1, s2, s3, s4, isem):
    rows = (r0, r1, r2, r3, r4)[:NBUF]
    gsem = (g0, g1, g2, g3, g4)[:NBUF]
    ssem = (s0, s1, s2, s3, s4)[:NBUF]
    cid = lax.axis_index("c")
    sid = lax.axis_index("s")
    wid = cid * NS + sid
    row0 = sid * ROWS_PER_TILE
    ebase = wid * EPT

    pltpu.async_copy(src_hbm.at[pl.ds(ebase, EPT)], sall, isem)
    pltpu.async_copy(dst_hbm.at[pl.ds(ebase, EPT)], dall, isem)

    zbuf = rows[0]
    zvec = jnp.zeros((16,), jnp.float32)

    def _zfill(t, _):
        zbuf[t // 8, pl.ds((t % 8) * 16, 16)] = zvec
        return 0

    lax.fori_loop(0, CHUNK * (D_OUT // 16), _zfill, 0)
    for k in range(ROWS_PER_TILE // CHUNK):
        pltpu.sync_copy(zbuf, acc_sh.at[pl.ds(row0 + k * CHUNK, CHUNK)])
    zrem = ROWS_PER_TILE % CHUNK
    if zrem:
        pltpu.sync_copy(
            zbuf.at[pl.ds(0, zrem)],
            acc_sh.at[pl.ds(row0 + (ROWS_PER_TILE // CHUNK) * CHUNK, zrem)])

    pltpu.make_async_copy(src_hbm.at[pl.ds(ebase, EPT)], sall, isem).wait()
    pltpu.make_async_copy(dst_hbm.at[pl.ds(ebase, EPT)], dall, isem).wait()
    plsc.subcore_barrier()

    def _sl(g):
        return pl.ds(lax.mul(g, CHUNK), CHUNK)

    def _gather(g, b):
        pltpu.async_copy(h_hbm.at[sall.at[_sl(g)]], rows[b], gsem[b])

    for b in range(NBUF):
        _gather(b, b)

    def _proc(g, b, refill):
        pltpu.make_async_copy(
            h_hbm.at[sall.at[_sl(g)]], rows[b], gsem[b]).wait()
        pltpu.async_copy(rows[b], acc_sh.at[dall.at[_sl(g)]], ssem[b],
                         add=True)
        if refill:
            @pl.when(g + NBUF < CPT)
            def _():
                pltpu.make_async_copy(
                    rows[b], acc_sh.at[dall.at[_sl(g)]], ssem[b]).wait()
                _gather(g + NBUF, b)
        else:
            pltpu.make_async_copy(
                rows[b], acc_sh.at[dall.at[_sl(g)]], ssem[b]).wait()

    def _step(t, _):
        for b in range(NBUF):
            _proc(t * NBUF + b, b, True)
        return 0

    lax.fori_loop(0, CPT // NBUF, _step, 0)
    for g in range((CPT // NBUF) * NBUF, CPT):
        _proc(g, g % NBUF, False)
    for g in range(max(0, CPT - NBUF), (CPT // NBUF) * NBUF):
        b = g % NBUF
        pltpu.make_async_copy(
            rows[b], acc_sh.at[dall.at[_sl(g)]], ssem[b]).wait()
    if TAIL:
        toff = pl.ds(CPT * CHUNK, TAIL)
        tbuf = rows[0].at[pl.ds(0, TAIL)]
        pltpu.async_copy(h_hbm.at[sall.at[toff]], tbuf, gsem[0])
        pltpu.make_async_copy(h_hbm.at[sall.at[toff]], tbuf, gsem[0]).wait()
        pltpu.sync_copy(tbuf, acc_sh.at[dall.at[toff]], add=True)
    plsc.subcore_barrier()

    pltpu.sync_copy(
        acc_sh.at[pl.ds(row0, ROWS_PER_TILE)],
        part_hbm.at[cid, pl.ds(row0, ROWS_PER_TILE)],
    )


_sc_aggregate = functools.partial(
    pl.kernel,
    out_type=jax.ShapeDtypeStruct((NC, N_PAD, D_OUT), jnp.float32),
    mesh=plsc.VectorSubcoreMesh(core_axis_name="c", subcore_axis_name="s"),
    scratch_types=[
        pltpu.VMEM((EPT,), jnp.int32),
        pltpu.VMEM((EPT,), jnp.int32),
        pltpu.VMEM((CHUNK, D_OUT), jnp.float32),
        pltpu.VMEM((CHUNK, D_OUT), jnp.float32),
        pltpu.VMEM((CHUNK, D_OUT), jnp.float32),
        pltpu.VMEM((CHUNK, D_OUT), jnp.float32),
        pltpu.VMEM((CHUNK, D_OUT), jnp.float32),
        pltpu.VMEM_SHARED((N_PAD, D_OUT), jnp.float32),
        pltpu.SemaphoreType.DMA,
        pltpu.SemaphoreType.DMA,
        pltpu.SemaphoreType.DMA,
        pltpu.SemaphoreType.DMA,
        pltpu.SemaphoreType.DMA,
        pltpu.SemaphoreType.DMA,
        pltpu.SemaphoreType.DMA,
        pltpu.SemaphoreType.DMA,
        pltpu.SemaphoreType.DMA,
        pltpu.SemaphoreType.DMA,
        pltpu.SemaphoreType.DMA,
    ],
)(_sc_body)


def kernel(x, adj, W, b):
    partials = _sc_aggregate(x, adj[0], adj[1])
    return _final(partials, W, b.reshape(1, D_OUT))
